# Initial kernel scaffold; baseline (speedup 1.0000x reference)
import jax
import jax.numpy as jnp
from jax.experimental import pallas as pl


def kernel(hidden_states, router_weight, router_bias, gate_up_proj, gate_up_bias, down_proj, down_bias):
    raise NotImplementedError("write your pallas kernel here")


# speedup vs baseline: 1.9593x; 1.9593x over previous
import functools

import jax
import jax.numpy as jnp
from jax.experimental import pallas as pl
from jax.experimental.pallas import tpu as pltpu

S = 2048
H = 1024
E = 8
K = 2
I = 1024
ALPHA = 1.702
LIMIT = 7.0

TOKEN_CHUNK = 512


def _router_body(x_ref, wrt_ref, b_ref, wd_ref):
    logits = jnp.dot(
        x_ref[...], wrt_ref[...],
        preferred_element_type=jnp.float32,
    ) + b_ref[...]
    m = jnp.max(logits, axis=-1, keepdims=True)
    ex = jnp.exp(logits - m)
    scores = ex / jnp.sum(ex, axis=-1, keepdims=True)

    iota = jax.lax.broadcasted_iota(jnp.int32, scores.shape, 1)
    v1 = jnp.max(scores, axis=-1, keepdims=True)
    i1 = jnp.min(jnp.where(scores == v1, iota, E), axis=-1, keepdims=True)
    masked = jnp.where(iota == i1, -jnp.inf, scores)
    v2 = jnp.max(masked, axis=-1, keepdims=True)
    i2 = jnp.min(jnp.where(masked == v2, iota, E), axis=-1, keepdims=True)

    wd = jnp.where(iota == i1, v1, jnp.where(iota == i2, v2, 0.0))
    wd_ref[...] = wd


def _ffn_body(x_ref, wg_ref, gub_ref, wdp_ref, db_ref, wd_ref, out_ref):
    e = pl.program_id(0)
    eiota = jax.lax.broadcasted_iota(jnp.int32, (1, E), 1)
    onehot = (eiota == e).astype(jnp.float32)

    for tc in range(S // TOKEN_CHUNK):
        rows = pl.ds(tc * TOKEN_CHUNK, TOKEN_CHUNK)
        xc = x_ref[rows, :]
        gu = jnp.dot(xc, wg_ref[0], preferred_element_type=jnp.float32)
        gu = gu + gub_ref[0]
        gate = gu[:, :I]
        up = gu[:, I:]
        gate = jnp.minimum(gate, LIMIT)
        up = jnp.clip(up, -LIMIT, LIMIT)
        glu = gate * jax.nn.sigmoid(gate * ALPHA)
        act = (up + 1.0) * glu
        y = jnp.dot(act.astype(jnp.bfloat16), wdp_ref[0],
                    preferred_element_type=jnp.float32)
        y = y + db_ref[0]
        w8 = wd_ref[rows, :]
        wcol = jnp.sum(w8 * onehot, axis=-1, keepdims=True)
        contrib = y * wcol

        @pl.when(e == 0)
        def _():
            out_ref[rows, :] = contrib

        @pl.when(e != 0)
        def _():
            out_ref[rows, :] = out_ref[rows, :] + contrib


def _router_call(flat, wrt, bias2d, interpret=False):
    return pl.pallas_call(
        _router_body,
        out_shape=jax.ShapeDtypeStruct((S, E), jnp.float32),
        interpret=interpret,
    )(flat, wrt, bias2d)


def _ffn_call(x_bf16, wg, gub, wdp, db, wd, interpret=False):
    return pl.pallas_call(
        _ffn_body,
        grid=(E,),
        in_specs=[
            pl.BlockSpec((S, H), lambda e: (0, 0)),
            pl.BlockSpec((1, H, 2 * I), lambda e: (e, 0, 0)),
            pl.BlockSpec((1, 1, 2 * I), lambda e: (e, 0, 0)),
            pl.BlockSpec((1, I, H), lambda e: (e, 0, 0)),
            pl.BlockSpec((1, 1, H), lambda e: (e, 0, 0)),
            pl.BlockSpec((S, E), lambda e: (0, 0)),
        ],
        out_specs=pl.BlockSpec((S, H), lambda e: (0, 0)),
        out_shape=jax.ShapeDtypeStruct((S, H), jnp.float32),
        interpret=interpret,
    )(x_bf16, wg, gub, wdp, db, wd)


@functools.partial(jax.jit, static_argnames=("interpret",))
def _kernel_impl(hidden_states, router_weight, router_bias, gate_up_proj,
                 gate_up_bias, down_proj, down_bias, interpret=False):
    batch, seq, hid = hidden_states.shape
    flat = hidden_states.reshape(batch * seq, hid)
    wrt = router_weight.T
    bias2d = router_bias.reshape(1, E)

    wd = _router_call(flat, wrt, bias2d, interpret=interpret)

    x_bf16 = flat.astype(jnp.bfloat16)
    wg = gate_up_proj.astype(jnp.bfloat16)
    wdp = down_proj.astype(jnp.bfloat16)
    gub3 = gate_up_bias.reshape(E, 1, 2 * I)
    db3 = down_bias.reshape(E, 1, H)
    out = _ffn_call(x_bf16, wg, gub3, wdp, db3, wd,
                    interpret=interpret)
    return out.reshape(batch, seq, hid)


def kernel(hidden_states, router_weight, router_bias, gate_up_proj,
           gate_up_bias, down_proj, down_bias):
    return _kernel_impl(hidden_states, router_weight, router_bias,
                        gate_up_proj, gate_up_bias, down_proj, down_bias)


# speedup vs baseline: 1.9800x; 1.0106x over previous
import functools

import jax
import jax.numpy as jnp
from jax import lax
from jax.experimental import pallas as pl
from jax.experimental.pallas import tpu as pltpu
from jax.experimental.pallas import tpu_sc as plsc

S = 2048
H = 1024
E = 8
K = 2
I = 1024
ALPHA = 1.702
LIMIT = 7.0

TILE = 128
P = S * K + E * TILE
NT = P // TILE
NTPAD = 64
CH = 256
NCH = (S * K) // CH

NSUB = 32
TOK_PER_SUB = S // NSUB
CCHUNK = 32


def _meta_body(x_ref, wrt_ref, b_ref,
               pos_ref, w0_ref, w1_ref, gid_ref, act_ref,
               oh_s, rank_s):
    logits = jnp.dot(x_ref[...], wrt_ref[...],
                     preferred_element_type=jnp.float32) + b_ref[...]
    m = jnp.max(logits, axis=-1, keepdims=True)
    ex = jnp.exp(logits - m)
    scores = ex / jnp.sum(ex, axis=-1, keepdims=True)

    iota = lax.broadcasted_iota(jnp.int32, scores.shape, 1)
    v1 = jnp.max(scores, axis=-1, keepdims=True)
    i1 = jnp.min(jnp.where(scores == v1, iota, E), axis=-1, keepdims=True)
    masked = jnp.where(iota == i1, -jnp.inf, scores)
    v2 = jnp.max(masked, axis=-1, keepdims=True)
    i2 = jnp.min(jnp.where(masked == v2, iota, E), axis=-1, keepdims=True)

    w0_ref[...] = jnp.broadcast_to(v1, (S, 16))
    w1_ref[...] = jnp.broadcast_to(v2, (S, 16))

    oh1 = (iota == i1).astype(jnp.bfloat16)
    oh2 = (iota == i2).astype(jnp.bfloat16)
    oh_s[...] = jnp.concatenate([oh1, oh2], axis=0)

    r = lax.broadcasted_iota(jnp.int32, (CH, CH), 0)
    c = lax.broadcasted_iota(jnp.int32, (CH, CH), 1)
    tril = (r > c).astype(jnp.bfloat16)

    def chunk_step(ci, base):
        rows = pl.ds(ci * CH, CH)
        ohc = oh_s[rows, :]
        within = jnp.dot(tril, ohc, preferred_element_type=jnp.float32)
        rank_s[rows, :] = within + base
        tot = within[CH - 1:CH, :] + ohc[CH - 1:CH, :].astype(jnp.float32)
        return base + tot

    counts = lax.fori_loop(0, NCH, chunk_step,
                           jnp.zeros((1, E), jnp.float32))

    ft = jnp.float32(TILE)
    aligned = jnp.floor((counts + (ft - 1.0)) / ft) * ft
    inc = aligned
    for sh in (1, 2, 4):
        inc = inc + jnp.concatenate(
            [jnp.zeros((1, sh), jnp.float32), inc[:, :-sh]], axis=1)
    off_end = inc
    off_start = off_end - aligned

    tstart = lax.broadcasted_iota(jnp.int32, (NTPAD, 1), 0).astype(
        jnp.float32) * ft
    ge = (tstart >= off_end).astype(jnp.int32)
    gid = jnp.minimum(jnp.sum(ge, axis=1, keepdims=True), E - 1)
    total = jnp.max(off_end, axis=1, keepdims=True)
    gid_ref[...] = gid
    act_ref[...] = (tstart < total).astype(jnp.int32)

    def pos_step(ci, carry):
        rows = pl.ds(ci * CH, CH)
        ohc = oh_s[rows, :].astype(jnp.float32)
        slot = jnp.sum((rank_s[rows, :] + off_start) * ohc,
                       axis=1, keepdims=True)
        pos_ref[rows, :] = slot.astype(jnp.int32)
        return carry

    lax.fori_loop(0, NCH, pos_step, jnp.int32(0))


def _meta_call(flat, wrt, bias2d):
    return pl.pallas_call(
        _meta_body,
        out_shape=[
            jax.ShapeDtypeStruct((S * K, 1), jnp.int32),
            jax.ShapeDtypeStruct((S, 16), jnp.float32),
            jax.ShapeDtypeStruct((S, 16), jnp.float32),
            jax.ShapeDtypeStruct((NTPAD, 1), jnp.int32),
            jax.ShapeDtypeStruct((NTPAD, 1), jnp.int32),
        ],
        scratch_shapes=[
            pltpu.VMEM((S * K, E), jnp.bfloat16),
            pltpu.VMEM((S * K, E), jnp.float32),
        ],
    )(flat, wrt, bias2d)


def _dispatch_body(x_hbm, pos_hbm, xg_hbm, rows_v, idx0_v, idx1_v, sem):
    wid = lax.axis_index("s") * 2 + lax.axis_index("c")
    base = wid * TOK_PER_SUB
    cp = pltpu.async_copy(x_hbm.at[pl.ds(base, TOK_PER_SUB)], rows_v, sem)
    pltpu.sync_copy(pos_hbm.at[0, wid], idx0_v)
    pltpu.sync_copy(pos_hbm.at[1, wid], idx1_v)
    cp.wait()
    pltpu.sync_copy(rows_v, xg_hbm.at[idx0_v])
    pltpu.sync_copy(rows_v, xg_hbm.at[idx1_v])


def _dispatch_call(flat, pos3):
    mesh = plsc.VectorSubcoreMesh(core_axis_name="c", subcore_axis_name="s")
    kern = pl.kernel(
        _dispatch_body,
        out_type=jax.ShapeDtypeStruct((P, H), jnp.float32),
        mesh=mesh,
        scratch_types=[
            pltpu.VMEM((TOK_PER_SUB, H), jnp.float32),
            pltpu.VMEM((TOK_PER_SUB,), jnp.int32),
            pltpu.VMEM((TOK_PER_SUB,), jnp.int32),
            pltpu.SemaphoreType.DMA,
        ],
    )
    return kern(flat, pos3)


def _gmm_body(gid_ref, act_ref, xg_ref, wg_ref, gub_ref, wdp_ref, db_ref,
              y_ref):
    i = pl.program_id(0)

    @pl.when(act_ref[i] == 1)
    def _():
        x = xg_ref[...].astype(jnp.bfloat16)
        gu = jnp.dot(x, wg_ref[0], preferred_element_type=jnp.float32)
        gu = gu + gub_ref[0]
        gate = jnp.minimum(gu[:, :I], LIMIT)
        up = jnp.clip(gu[:, I:], -LIMIT, LIMIT)
        glu = gate * jax.nn.sigmoid(gate * ALPHA)
        act = (up + 1.0) * glu
        y = jnp.dot(act.astype(jnp.bfloat16), wdp_ref[0],
                    preferred_element_type=jnp.float32)
        y_ref[...] = y + db_ref[0]


def _gmm_call(xg, wg, gub3, wdp, db3, gid, actv):
    grid_spec = pltpu.PrefetchScalarGridSpec(
        num_scalar_prefetch=2,
        grid=(NT,),
        in_specs=[
            pl.BlockSpec((TILE, H), lambda i, g, a: (i, 0)),
            pl.BlockSpec((1, H, 2 * I), lambda i, g, a: (g[i], 0, 0)),
            pl.BlockSpec((1, 1, 2 * I), lambda i, g, a: (g[i], 0, 0)),
            pl.BlockSpec((1, I, H), lambda i, g, a: (g[i], 0, 0)),
            pl.BlockSpec((1, 1, H), lambda i, g, a: (g[i], 0, 0)),
        ],
        out_specs=pl.BlockSpec((TILE, H), lambda i, g, a: (i, 0)),
    )
    return pl.pallas_call(
        _gmm_body,
        grid_spec=grid_spec,
        out_shape=jax.ShapeDtypeStruct((P, H), jnp.float32),
    )(gid, actv, xg, wg, gub3, wdp, db3)


def _combine_body(y_hbm, pos_hbm, w0_hbm, w1_hbm, out_hbm,
                  buf0, buf1, idx0_v, idx1_v, w0_v, w1_v, sem0, sem1):
    wid = lax.axis_index("s") * 2 + lax.axis_index("c")

    @pl.loop(0, TOK_PER_SUB // CCHUNK)
    def _(ci):
        tok = wid * TOK_PER_SUB + ci * CCHUNK
        pltpu.sync_copy(pos_hbm.at[0, pl.ds(tok, CCHUNK)], idx0_v)
        pltpu.sync_copy(pos_hbm.at[1, pl.ds(tok, CCHUNK)], idx1_v)
        cp0 = pltpu.async_copy(y_hbm.at[idx0_v], buf0, sem0)
        cp1 = pltpu.async_copy(y_hbm.at[idx1_v], buf1, sem1)
        pltpu.sync_copy(w0_hbm.at[pl.ds(tok, CCHUNK)], w0_v)
        pltpu.sync_copy(w1_hbm.at[pl.ds(tok, CCHUNK)], w1_v)
        cp0.wait()
        cp1.wait()

        @pl.loop(0, CCHUNK)
        def _(t):
            w0 = w0_v[t, :]
            w1 = w1_v[t, :]

            @pl.loop(0, H, step=16)
            def _(h):
                buf0[t, pl.ds(h, 16)] = (
                    w0 * buf0[t, pl.ds(h, 16)] + w1 * buf1[t, pl.ds(h, 16)])

        pltpu.sync_copy(buf0, out_hbm.at[pl.ds(tok, CCHUNK)])


def _combine_call(y, pos2, w0w, w1w):
    mesh = plsc.VectorSubcoreMesh(core_axis_name="c", subcore_axis_name="s")
    kern = pl.kernel(
        _combine_body,
        out_type=jax.ShapeDtypeStruct((S, H), jnp.float32),
        mesh=mesh,
        scratch_types=[
            pltpu.VMEM((CCHUNK, H), jnp.float32),
            pltpu.VMEM((CCHUNK, H), jnp.float32),
            pltpu.VMEM((CCHUNK,), jnp.int32),
            pltpu.VMEM((CCHUNK,), jnp.int32),
            pltpu.VMEM((CCHUNK, 16), jnp.float32),
            pltpu.VMEM((CCHUNK, 16), jnp.float32),
            pltpu.SemaphoreType.DMA,
            pltpu.SemaphoreType.DMA,
        ],
    )
    return kern(y, pos2, w0w, w1w)


@jax.jit
def _kernel_impl(hidden_states, router_weight, router_bias, gate_up_proj,
                 gate_up_bias, down_proj, down_bias):
    batch, seq, hid = hidden_states.shape
    flat = hidden_states.reshape(batch * seq, hid)
    wrt = router_weight.T
    bias2d = router_bias.reshape(1, E)

    pos, w0w, w1w, gid, actv = _meta_call(flat, wrt, bias2d)
    pos3 = pos.reshape(K, NSUB, TOK_PER_SUB)
    gid1 = gid.reshape(NTPAD)
    actv1 = actv.reshape(NTPAD)

    xg = _dispatch_call(flat, pos3)

    wg = gate_up_proj.astype(jnp.bfloat16)
    wdp = down_proj.astype(jnp.bfloat16)
    gub3 = gate_up_bias.reshape(E, 1, 2 * I)
    db3 = down_bias.reshape(E, 1, H)
    y = _gmm_call(xg, wg, gub3, wdp, db3, gid1, actv1)

    pos2 = pos.reshape(K, S)
    out = _combine_call(y, pos2, w0w, w1w)
    return out.reshape(batch, seq, hid)


def kernel(hidden_states, router_weight, router_bias, gate_up_proj,
           gate_up_bias, down_proj, down_bias):
    return _kernel_impl(hidden_states, router_weight, router_bias,
                        gate_up_proj, gate_up_bias, down_proj, down_bias)


# speedup vs baseline: 2.3630x; 1.1934x over previous
import functools

import jax
import jax.numpy as jnp
from jax import lax
from jax.experimental import pallas as pl
from jax.experimental.pallas import tpu as pltpu
from jax.experimental.pallas import tpu_sc as plsc

S = 2048
H = 1024
E = 8
K = 2
I = 1024
ALPHA = 1.702
LIMIT = 7.0

TILE = 128
P = S * K + E * TILE
NT = P // TILE
NTPAD = 64
CH = 256
NCH = (S * K) // CH

NSUB = 32
TOK_PER_SUB = S // NSUB
CCHUNK = 32


def _meta_body(x_ref, wrt_ref, b_ref,
               pos_ref, w0_ref, w1_ref, gid_ref, act_ref,
               oh_s, rank_s):
    logits = jnp.dot(x_ref[...], wrt_ref[...],
                     preferred_element_type=jnp.float32) + b_ref[...]
    m = jnp.max(logits, axis=-1, keepdims=True)
    ex = jnp.exp(logits - m)
    scores = ex / jnp.sum(ex, axis=-1, keepdims=True)

    iota = lax.broadcasted_iota(jnp.int32, scores.shape, 1)
    v1 = jnp.max(scores, axis=-1, keepdims=True)
    i1 = jnp.min(jnp.where(scores == v1, iota, E), axis=-1, keepdims=True)
    masked = jnp.where(iota == i1, -jnp.inf, scores)
    v2 = jnp.max(masked, axis=-1, keepdims=True)
    i2 = jnp.min(jnp.where(masked == v2, iota, E), axis=-1, keepdims=True)

    w0_ref[...] = jnp.broadcast_to(v1, (S, 16))
    w1_ref[...] = jnp.broadcast_to(v2, (S, 16))

    oh1 = (iota == i1).astype(jnp.bfloat16)
    oh2 = (iota == i2).astype(jnp.bfloat16)
    oh_s[...] = jnp.concatenate([oh1, oh2], axis=0)

    r = lax.broadcasted_iota(jnp.int32, (CH, CH), 0)
    c = lax.broadcasted_iota(jnp.int32, (CH, CH), 1)
    tril = (r > c).astype(jnp.bfloat16)

    def chunk_step(ci, base):
        rows = pl.ds(ci * CH, CH)
        ohc = oh_s[rows, :]
        within = jnp.dot(tril, ohc, preferred_element_type=jnp.float32)
        rank_s[rows, :] = within + base
        tot = within[CH - 1:CH, :] + ohc[CH - 1:CH, :].astype(jnp.float32)
        return base + tot

    counts = lax.fori_loop(0, NCH, chunk_step,
                           jnp.zeros((1, E), jnp.float32))

    ft = jnp.float32(TILE)
    aligned = jnp.floor((counts + (ft - 1.0)) / ft) * ft
    inc = aligned
    for sh in (1, 2, 4):
        inc = inc + jnp.concatenate(
            [jnp.zeros((1, sh), jnp.float32), inc[:, :-sh]], axis=1)
    off_end = inc
    off_start = off_end - aligned

    tstart = lax.broadcasted_iota(jnp.int32, (NTPAD, 1), 0).astype(
        jnp.float32) * ft
    ge = (tstart >= off_end).astype(jnp.int32)
    gid = jnp.minimum(jnp.sum(ge, axis=1, keepdims=True), E - 1)
    total = jnp.max(off_end, axis=1, keepdims=True)
    gid_ref[...] = gid
    act_ref[...] = (tstart < total).astype(jnp.int32)

    def pos_step(ci, carry):
        rows = pl.ds(ci * CH, CH)
        ohc = oh_s[rows, :].astype(jnp.float32)
        slot = jnp.sum((rank_s[rows, :] + off_start) * ohc,
                       axis=1, keepdims=True)
        pos_ref[rows, :] = slot.astype(jnp.int32)
        return carry

    lax.fori_loop(0, NCH, pos_step, jnp.int32(0))


def _meta_call(flat, wrt, bias2d):
    return pl.pallas_call(
        _meta_body,
        out_shape=[
            jax.ShapeDtypeStruct((S * K, 1), jnp.int32),
            jax.ShapeDtypeStruct((S, 16), jnp.float32),
            jax.ShapeDtypeStruct((S, 16), jnp.float32),
            jax.ShapeDtypeStruct((NTPAD, 1), jnp.int32),
            jax.ShapeDtypeStruct((NTPAD, 1), jnp.int32),
        ],
        scratch_shapes=[
            pltpu.VMEM((S * K, E), jnp.bfloat16),
            pltpu.VMEM((S * K, E), jnp.float32),
        ],
    )(flat, wrt, bias2d)


def _dispatch_body(x_hbm, pos_hbm, xg_hbm, rows_v, idx0_v, idx1_v, sem):
    wid = lax.axis_index("s") * 2 + lax.axis_index("c")
    base = wid * TOK_PER_SUB
    cp = pltpu.async_copy(x_hbm.at[pl.ds(base, TOK_PER_SUB)], rows_v, sem)
    pltpu.sync_copy(pos_hbm.at[0, wid], idx0_v)
    pltpu.sync_copy(pos_hbm.at[1, wid], idx1_v)
    cp.wait()
    pltpu.sync_copy(rows_v, xg_hbm.at[idx0_v])
    pltpu.sync_copy(rows_v, xg_hbm.at[idx1_v])


def _dispatch_call(flat, pos3):
    mesh = plsc.VectorSubcoreMesh(core_axis_name="c", subcore_axis_name="s")
    kern = pl.kernel(
        _dispatch_body,
        out_type=jax.ShapeDtypeStruct((P, H), jnp.float32),
        mesh=mesh,
        scratch_types=[
            pltpu.VMEM((TOK_PER_SUB, H), jnp.float32),
            pltpu.VMEM((TOK_PER_SUB,), jnp.int32),
            pltpu.VMEM((TOK_PER_SUB,), jnp.int32),
            pltpu.SemaphoreType.DMA,
        ],
    )
    return kern(flat, pos3)


def _gmm_body(gid_ref, act_ref, xg_ref, wg_ref, gub_ref, wdp_ref, db_ref,
              y_ref):
    i = pl.program_id(0)

    @pl.when(act_ref[i] == 1)
    def _():
        x = xg_ref[...]
        gu = jnp.dot(x, wg_ref[0], preferred_element_type=jnp.float32)
        gu = gu + gub_ref[0]
        gate = jnp.minimum(gu[:, :I], LIMIT)
        up = jnp.clip(gu[:, I:], -LIMIT, LIMIT)
        glu = gate * jax.nn.sigmoid(gate * ALPHA)
        act = (up + 1.0) * glu
        y = jnp.dot(act, wdp_ref[0], preferred_element_type=jnp.float32)
        y_ref[...] = y + db_ref[0]


def _gmm_call(xg, wg, gub3, wdp, db3, gid, actv):
    grid_spec = pltpu.PrefetchScalarGridSpec(
        num_scalar_prefetch=2,
        grid=(NT,),
        in_specs=[
            pl.BlockSpec((TILE, H), lambda i, g, a: (i, 0)),
            pl.BlockSpec((1, H, 2 * I), lambda i, g, a: (g[i], 0, 0)),
            pl.BlockSpec((1, 1, 2 * I), lambda i, g, a: (g[i], 0, 0)),
            pl.BlockSpec((1, I, H), lambda i, g, a: (g[i], 0, 0)),
            pl.BlockSpec((1, 1, H), lambda i, g, a: (g[i], 0, 0)),
        ],
        out_specs=pl.BlockSpec((TILE, H), lambda i, g, a: (i, 0)),
    )
    return pl.pallas_call(
        _gmm_body,
        grid_spec=grid_spec,
        out_shape=jax.ShapeDtypeStruct((P, H), jnp.float32),
    )(gid, actv, xg, wg, gub3, wdp, db3)


def _combine_body(y_hbm, pos_hbm, w0_hbm, w1_hbm, out_hbm,
                  buf0, buf1, idx0_v, idx1_v, w0_v, w1_v, sem0, sem1):
    wid = lax.axis_index("s") * 2 + lax.axis_index("c")

    @pl.loop(0, TOK_PER_SUB // CCHUNK)
    def _(ci):
        tok = wid * TOK_PER_SUB + ci * CCHUNK
        pltpu.sync_copy(pos_hbm.at[0, pl.ds(tok, CCHUNK)], idx0_v)
        pltpu.sync_copy(pos_hbm.at[1, pl.ds(tok, CCHUNK)], idx1_v)
        cp0 = pltpu.async_copy(y_hbm.at[idx0_v], buf0, sem0)
        cp1 = pltpu.async_copy(y_hbm.at[idx1_v], buf1, sem1)
        pltpu.sync_copy(w0_hbm.at[pl.ds(tok, CCHUNK)], w0_v)
        pltpu.sync_copy(w1_hbm.at[pl.ds(tok, CCHUNK)], w1_v)
        cp0.wait()
        cp1.wait()

        @pl.loop(0, CCHUNK)
        def _(t):
            w0 = w0_v[t, :]
            w1 = w1_v[t, :]

            @pl.loop(0, H, step=16)
            def _(h):
                buf0[t, pl.ds(h, 16)] = (
                    w0 * buf0[t, pl.ds(h, 16)] + w1 * buf1[t, pl.ds(h, 16)])

        pltpu.sync_copy(buf0, out_hbm.at[pl.ds(tok, CCHUNK)])


def _combine_call(y, pos2, w0w, w1w):
    mesh = plsc.VectorSubcoreMesh(core_axis_name="c", subcore_axis_name="s")
    kern = pl.kernel(
        _combine_body,
        out_type=jax.ShapeDtypeStruct((S, H), jnp.float32),
        mesh=mesh,
        scratch_types=[
            pltpu.VMEM((CCHUNK, H), jnp.float32),
            pltpu.VMEM((CCHUNK, H), jnp.float32),
            pltpu.VMEM((CCHUNK,), jnp.int32),
            pltpu.VMEM((CCHUNK,), jnp.int32),
            pltpu.VMEM((CCHUNK, 16), jnp.float32),
            pltpu.VMEM((CCHUNK, 16), jnp.float32),
            pltpu.SemaphoreType.DMA,
            pltpu.SemaphoreType.DMA,
        ],
    )
    return kern(y, pos2, w0w, w1w)


@jax.jit
def _kernel_impl(hidden_states, router_weight, router_bias, gate_up_proj,
                 gate_up_bias, down_proj, down_bias):
    batch, seq, hid = hidden_states.shape
    flat = hidden_states.reshape(batch * seq, hid)
    wrt = router_weight.T
    bias2d = router_bias.reshape(1, E)

    pos, w0w, w1w, gid, actv = _meta_call(flat, wrt, bias2d)
    pos3 = pos.reshape(K, NSUB, TOK_PER_SUB)
    gid1 = gid.reshape(NTPAD)
    actv1 = actv.reshape(NTPAD)

    xg = _dispatch_call(flat, pos3)

    gub3 = gate_up_bias.reshape(E, 1, 2 * I)
    db3 = down_bias.reshape(E, 1, H)
    y = _gmm_call(xg, gate_up_proj, gub3, down_proj, db3, gid1, actv1)

    pos2 = pos.reshape(K, S)
    out = _combine_call(y, pos2, w0w, w1w)
    return out.reshape(batch, seq, hid)


def kernel(hidden_states, router_weight, router_bias, gate_up_proj,
           gate_up_bias, down_proj, down_bias):
    return _kernel_impl(hidden_states, router_weight, router_bias,
                        gate_up_proj, gate_up_bias, down_proj, down_bias)


# speedup vs baseline: 2.5113x; 1.0628x over previous
import functools

import jax
import jax.numpy as jnp
from jax import lax
from jax.experimental import pallas as pl
from jax.experimental.pallas import tpu as pltpu
from jax.experimental.pallas import tpu_sc as plsc

S = 2048
H = 1024
E = 8
K = 2
I = 1024
ALPHA = 1.702
LIMIT = 7.0

TILE = 256
P = S * K + E * TILE
NT = P // TILE
NTPAD = 64
CH = 256
NCH = (S * K) // CH

NSUB = 32
TOK_PER_SUB = S // NSUB
CCHUNK = 32


def _meta_body(x_ref, wrt_ref, b_ref,
               pos_ref, w0_ref, w1_ref, gid_ref, act_ref,
               oh_s, rank_s):
    logits = jnp.dot(x_ref[...], wrt_ref[...],
                     preferred_element_type=jnp.float32) + b_ref[...]
    m = jnp.max(logits, axis=-1, keepdims=True)
    ex = jnp.exp(logits - m)
    scores = ex / jnp.sum(ex, axis=-1, keepdims=True)

    iota = lax.broadcasted_iota(jnp.int32, scores.shape, 1)
    v1 = jnp.max(scores, axis=-1, keepdims=True)
    i1 = jnp.min(jnp.where(scores == v1, iota, E), axis=-1, keepdims=True)
    masked = jnp.where(iota == i1, -jnp.inf, scores)
    v2 = jnp.max(masked, axis=-1, keepdims=True)
    i2 = jnp.min(jnp.where(masked == v2, iota, E), axis=-1, keepdims=True)

    w0_ref[...] = jnp.broadcast_to(v1, (S, 16))
    w1_ref[...] = jnp.broadcast_to(v2, (S, 16))

    oh1 = (iota == i1).astype(jnp.bfloat16)
    oh2 = (iota == i2).astype(jnp.bfloat16)
    oh_s[...] = jnp.concatenate([oh1, oh2], axis=0)

    r = lax.broadcasted_iota(jnp.int32, (CH, CH), 0)
    c = lax.broadcasted_iota(jnp.int32, (CH, CH), 1)
    tril = (r > c).astype(jnp.bfloat16)

    def chunk_step(ci, base):
        rows = pl.ds(ci * CH, CH)
        ohc = oh_s[rows, :]
        within = jnp.dot(tril, ohc, preferred_element_type=jnp.float32)
        rank_s[rows, :] = within + base
        tot = within[CH - 1:CH, :] + ohc[CH - 1:CH, :].astype(jnp.float32)
        return base + tot

    counts = lax.fori_loop(0, NCH, chunk_step,
                           jnp.zeros((1, E), jnp.float32))

    ft = jnp.float32(TILE)
    aligned = jnp.floor((counts + (ft - 1.0)) / ft) * ft
    inc = aligned
    for sh in (1, 2, 4):
        inc = inc + jnp.concatenate(
            [jnp.zeros((1, sh), jnp.float32), inc[:, :-sh]], axis=1)
    off_end = inc
    off_start = off_end - aligned

    tstart = lax.broadcasted_iota(jnp.int32, (NTPAD, 1), 0).astype(
        jnp.float32) * ft
    ge = (tstart >= off_end).astype(jnp.int32)
    gid = jnp.minimum(jnp.sum(ge, axis=1, keepdims=True), E - 1)
    total = jnp.max(off_end, axis=1, keepdims=True)
    gid_ref[...] = gid
    act_ref[...] = (tstart < total).astype(jnp.int32)

    def pos_step(ci, carry):
        rows = pl.ds(ci * CH, CH)
        ohc = oh_s[rows, :].astype(jnp.float32)
        slot = jnp.sum((rank_s[rows, :] + off_start) * ohc,
                       axis=1, keepdims=True)
        pos_ref[rows, :] = slot.astype(jnp.int32)
        return carry

    lax.fori_loop(0, NCH, pos_step, jnp.int32(0))


def _meta_call(flat, wrt, bias2d):
    return pl.pallas_call(
        _meta_body,
        out_shape=[
            jax.ShapeDtypeStruct((S * K, 1), jnp.int32),
            jax.ShapeDtypeStruct((S, 16), jnp.float32),
            jax.ShapeDtypeStruct((S, 16), jnp.float32),
            jax.ShapeDtypeStruct((NTPAD, 1), jnp.int32),
            jax.ShapeDtypeStruct((NTPAD, 1), jnp.int32),
        ],
        scratch_shapes=[
            pltpu.VMEM((S * K, E), jnp.bfloat16),
            pltpu.VMEM((S * K, E), jnp.float32),
        ],
    )(flat, wrt, bias2d)


def _dispatch_body(x_hbm, pos_hbm, xg_hbm, rows_v, idx0_v, idx1_v, sem):
    wid = lax.axis_index("s") * 2 + lax.axis_index("c")
    base = wid * TOK_PER_SUB
    cp = pltpu.async_copy(x_hbm.at[pl.ds(base, TOK_PER_SUB)], rows_v, sem)
    pltpu.sync_copy(pos_hbm.at[0, wid], idx0_v)
    pltpu.sync_copy(pos_hbm.at[1, wid], idx1_v)
    cp.wait()
    pltpu.sync_copy(rows_v, xg_hbm.at[idx0_v])
    pltpu.sync_copy(rows_v, xg_hbm.at[idx1_v])


def _dispatch_call(flat, pos3):
    mesh = plsc.VectorSubcoreMesh(core_axis_name="c", subcore_axis_name="s")
    kern = pl.kernel(
        _dispatch_body,
        out_type=jax.ShapeDtypeStruct((P, H), jnp.float32),
        mesh=mesh,
        scratch_types=[
            pltpu.VMEM((TOK_PER_SUB, H), jnp.float32),
            pltpu.VMEM((TOK_PER_SUB,), jnp.int32),
            pltpu.VMEM((TOK_PER_SUB,), jnp.int32),
            pltpu.SemaphoreType.DMA,
        ],
    )
    return kern(flat, pos3)


def _gmm_body(gid_ref, act_ref, xg_ref, wg_ref, gub_ref, wdp_ref, db_ref,
              y_ref):
    i = pl.program_id(0)

    @pl.when(act_ref[i] == 1)
    def _():
        x = xg_ref[...]
        gu = jnp.dot(x, wg_ref[0], preferred_element_type=jnp.float32)
        gu = gu + gub_ref[0]
        gate = jnp.minimum(gu[:, :I], LIMIT)
        up = jnp.clip(gu[:, I:], -LIMIT, LIMIT)
        glu = gate * jax.nn.sigmoid(gate * ALPHA)
        act = (up + 1.0) * glu
        y = jnp.dot(act, wdp_ref[0], preferred_element_type=jnp.float32)
        y_ref[...] = y + db_ref[0]


def _gmm_call(xg, wg, gub3, wdp, db3, gid, actv):
    grid_spec = pltpu.PrefetchScalarGridSpec(
        num_scalar_prefetch=2,
        grid=(NT,),
        in_specs=[
            pl.BlockSpec((TILE, H), lambda i, g, a: (i, 0)),
            pl.BlockSpec((1, H, 2 * I), lambda i, g, a: (g[i], 0, 0)),
            pl.BlockSpec((1, 1, 2 * I), lambda i, g, a: (g[i], 0, 0)),
            pl.BlockSpec((1, I, H), lambda i, g, a: (g[i], 0, 0)),
            pl.BlockSpec((1, 1, H), lambda i, g, a: (g[i], 0, 0)),
        ],
        out_specs=pl.BlockSpec((TILE, H), lambda i, g, a: (i, 0)),
    )
    return pl.pallas_call(
        _gmm_body,
        grid_spec=grid_spec,
        out_shape=jax.ShapeDtypeStruct((P, H), jnp.float32),
    )(gid, actv, xg, wg, gub3, wdp, db3)


def _combine_body(y_hbm, pos_hbm, w0_hbm, w1_hbm, out_hbm,
                  buf0, buf1, idx0_v, idx1_v, w0_v, w1_v, sem0, sem1):
    wid = lax.axis_index("s") * 2 + lax.axis_index("c")

    @pl.loop(0, TOK_PER_SUB // CCHUNK)
    def _(ci):
        tok = wid * TOK_PER_SUB + ci * CCHUNK
        pltpu.sync_copy(pos_hbm.at[0, pl.ds(tok, CCHUNK)], idx0_v)
        pltpu.sync_copy(pos_hbm.at[1, pl.ds(tok, CCHUNK)], idx1_v)
        cp0 = pltpu.async_copy(y_hbm.at[idx0_v], buf0, sem0)
        cp1 = pltpu.async_copy(y_hbm.at[idx1_v], buf1, sem1)
        pltpu.sync_copy(w0_hbm.at[pl.ds(tok, CCHUNK)], w0_v)
        pltpu.sync_copy(w1_hbm.at[pl.ds(tok, CCHUNK)], w1_v)
        cp0.wait()
        cp1.wait()

        @pl.loop(0, CCHUNK)
        def _(t):
            w0 = w0_v[t, :]
            w1 = w1_v[t, :]

            @pl.loop(0, H, step=16)
            def _(h):
                buf0[t, pl.ds(h, 16)] = (
                    w0 * buf0[t, pl.ds(h, 16)] + w1 * buf1[t, pl.ds(h, 16)])

        pltpu.sync_copy(buf0, out_hbm.at[pl.ds(tok, CCHUNK)])


def _combine_call(y, pos2, w0w, w1w):
    mesh = plsc.VectorSubcoreMesh(core_axis_name="c", subcore_axis_name="s")
    kern = pl.kernel(
        _combine_body,
        out_type=jax.ShapeDtypeStruct((S, H), jnp.float32),
        mesh=mesh,
        scratch_types=[
            pltpu.VMEM((CCHUNK, H), jnp.float32),
            pltpu.VMEM((CCHUNK, H), jnp.float32),
            pltpu.VMEM((CCHUNK,), jnp.int32),
            pltpu.VMEM((CCHUNK,), jnp.int32),
            pltpu.VMEM((CCHUNK, 16), jnp.float32),
            pltpu.VMEM((CCHUNK, 16), jnp.float32),
            pltpu.SemaphoreType.DMA,
            pltpu.SemaphoreType.DMA,
        ],
    )
    return kern(y, pos2, w0w, w1w)


@jax.jit
def _kernel_impl(hidden_states, router_weight, router_bias, gate_up_proj,
                 gate_up_bias, down_proj, down_bias):
    batch, seq, hid = hidden_states.shape
    flat = hidden_states.reshape(batch * seq, hid)
    wrt = router_weight.T
    bias2d = router_bias.reshape(1, E)

    pos, w0w, w1w, gid, actv = _meta_call(flat, wrt, bias2d)
    pos3 = pos.reshape(K, NSUB, TOK_PER_SUB)
    gid1 = gid.reshape(NTPAD)
    actv1 = actv.reshape(NTPAD)

    xg = _dispatch_call(flat, pos3)

    gub3 = gate_up_bias.reshape(E, 1, 2 * I)
    db3 = down_bias.reshape(E, 1, H)
    y = _gmm_call(xg, gate_up_proj, gub3, down_proj, db3, gid1, actv1)

    pos2 = pos.reshape(K, S)
    out = _combine_call(y, pos2, w0w, w1w)
    return out.reshape(batch, seq, hid)


def kernel(hidden_states, router_weight, router_bias, gate_up_proj,
           gate_up_bias, down_proj, down_bias):
    return _kernel_impl(hidden_states, router_weight, router_bias,
                        gate_up_proj, gate_up_bias, down_proj, down_bias)


# speedup vs baseline: 2.6934x; 1.0725x over previous
import functools

import jax
import jax.numpy as jnp
from jax import lax
from jax.experimental import pallas as pl
from jax.experimental.pallas import tpu as pltpu
from jax.experimental.pallas import tpu_sc as plsc

S = 2048
H = 1024
E = 8
K = 2
I = 1024
ALPHA = 1.702
LIMIT = 7.0

TILE = 512
P = S * K + E * TILE
NT = P // TILE
NTPAD = 64
CH = 256
NCH = (S * K) // CH

NSUB = 32
TOK_PER_SUB = S // NSUB
CCHUNK = 32


def _meta_body(x_ref, wrt_ref, b_ref,
               pos_ref, w0_ref, w1_ref, gid_ref, act_ref,
               oh_s, rank_s):
    logits = jnp.dot(x_ref[...], wrt_ref[...],
                     preferred_element_type=jnp.float32) + b_ref[...]
    m = jnp.max(logits, axis=-1, keepdims=True)
    ex = jnp.exp(logits - m)
    scores = ex / jnp.sum(ex, axis=-1, keepdims=True)

    iota = lax.broadcasted_iota(jnp.int32, scores.shape, 1)
    v1 = jnp.max(scores, axis=-1, keepdims=True)
    i1 = jnp.min(jnp.where(scores == v1, iota, E), axis=-1, keepdims=True)
    masked = jnp.where(iota == i1, -jnp.inf, scores)
    v2 = jnp.max(masked, axis=-1, keepdims=True)
    i2 = jnp.min(jnp.where(masked == v2, iota, E), axis=-1, keepdims=True)

    w0_ref[...] = jnp.broadcast_to(v1, (S, 16))
    w1_ref[...] = jnp.broadcast_to(v2, (S, 16))

    oh1 = (iota == i1).astype(jnp.bfloat16)
    oh2 = (iota == i2).astype(jnp.bfloat16)
    oh_s[...] = jnp.concatenate([oh1, oh2], axis=0)

    r = lax.broadcasted_iota(jnp.int32, (CH, CH), 0)
    c = lax.broadcasted_iota(jnp.int32, (CH, CH), 1)
    tril = (r > c).astype(jnp.bfloat16)

    def chunk_step(ci, base):
        rows = pl.ds(ci * CH, CH)
        ohc = oh_s[rows, :]
        within = jnp.dot(tril, ohc, preferred_element_type=jnp.float32)
        rank_s[rows, :] = within + base
        tot = within[CH - 1:CH, :] + ohc[CH - 1:CH, :].astype(jnp.float32)
        return base + tot

    counts = lax.fori_loop(0, NCH, chunk_step,
                           jnp.zeros((1, E), jnp.float32))

    ft = jnp.float32(TILE)
    aligned = jnp.floor((counts + (ft - 1.0)) / ft) * ft
    inc = aligned
    for sh in (1, 2, 4):
        inc = inc + jnp.concatenate(
            [jnp.zeros((1, sh), jnp.float32), inc[:, :-sh]], axis=1)
    off_end = inc
    off_start = off_end - aligned

    tstart = lax.broadcasted_iota(jnp.int32, (NTPAD, 1), 0).astype(
        jnp.float32) * ft
    ge = (tstart >= off_end).astype(jnp.int32)
    gid = jnp.minimum(jnp.sum(ge, axis=1, keepdims=True), E - 1)
    total = jnp.max(off_end, axis=1, keepdims=True)
    gid_ref[...] = gid
    act_ref[...] = (tstart < total).astype(jnp.int32)

    def pos_step(ci, carry):
        rows = pl.ds(ci * CH, CH)
        ohc = oh_s[rows, :].astype(jnp.float32)
        slot = jnp.sum((rank_s[rows, :] + off_start) * ohc,
                       axis=1, keepdims=True)
        pos_ref[rows, :] = slot.astype(jnp.int32)
        return carry

    lax.fori_loop(0, NCH, pos_step, jnp.int32(0))


def _meta_call(flat, wrt, bias2d):
    return pl.pallas_call(
        _meta_body,
        out_shape=[
            jax.ShapeDtypeStruct((S * K, 1), jnp.int32),
            jax.ShapeDtypeStruct((S, 16), jnp.float32),
            jax.ShapeDtypeStruct((S, 16), jnp.float32),
            jax.ShapeDtypeStruct((NTPAD, 1), jnp.int32),
            jax.ShapeDtypeStruct((NTPAD, 1), jnp.int32),
        ],
        scratch_shapes=[
            pltpu.VMEM((S * K, E), jnp.bfloat16),
            pltpu.VMEM((S * K, E), jnp.float32),
        ],
    )(flat, wrt, bias2d)


def _dispatch_body(x_hbm, pos_hbm, xg_hbm, rows_v, idx0_v, idx1_v, sem):
    wid = lax.axis_index("s") * 2 + lax.axis_index("c")
    base = wid * TOK_PER_SUB
    cp = pltpu.async_copy(x_hbm.at[pl.ds(base, TOK_PER_SUB)], rows_v, sem)
    pltpu.sync_copy(pos_hbm.at[0, wid], idx0_v)
    pltpu.sync_copy(pos_hbm.at[1, wid], idx1_v)
    cp.wait()
    pltpu.sync_copy(rows_v, xg_hbm.at[idx0_v])
    pltpu.sync_copy(rows_v, xg_hbm.at[idx1_v])


def _dispatch_call(flat, pos3):
    mesh = plsc.VectorSubcoreMesh(core_axis_name="c", subcore_axis_name="s")
    kern = pl.kernel(
        _dispatch_body,
        out_type=jax.ShapeDtypeStruct((P, H), jnp.float32),
        mesh=mesh,
        scratch_types=[
            pltpu.VMEM((TOK_PER_SUB, H), jnp.float32),
            pltpu.VMEM((TOK_PER_SUB,), jnp.int32),
            pltpu.VMEM((TOK_PER_SUB,), jnp.int32),
            pltpu.SemaphoreType.DMA,
        ],
    )
    return kern(flat, pos3)


def _gmm_body(gid_ref, act_ref, xg_ref, wg_ref, gub_ref, wdp_ref, db_ref,
              y_ref):
    i = pl.program_id(0)

    @pl.when(act_ref[i] == 1)
    def _():
        x = xg_ref[...]
        gu = jnp.dot(x, wg_ref[0], preferred_element_type=jnp.float32)
        gu = gu + gub_ref[0]
        gate = jnp.minimum(gu[:, :I], LIMIT)
        up = jnp.clip(gu[:, I:], -LIMIT, LIMIT)
        glu = gate * jax.nn.sigmoid(gate * ALPHA)
        act = (up + 1.0) * glu
        y = jnp.dot(act, wdp_ref[0], preferred_element_type=jnp.float32)
        y_ref[...] = y + db_ref[0]


def _gmm_call(xg, wg, gub3, wdp, db3, gid, actv):
    grid_spec = pltpu.PrefetchScalarGridSpec(
        num_scalar_prefetch=2,
        grid=(NT,),
        in_specs=[
            pl.BlockSpec((TILE, H), lambda i, g, a: (i, 0)),
            pl.BlockSpec((1, H, 2 * I), lambda i, g, a: (g[i], 0, 0)),
            pl.BlockSpec((1, 1, 2 * I), lambda i, g, a: (g[i], 0, 0)),
            pl.BlockSpec((1, I, H), lambda i, g, a: (g[i], 0, 0)),
            pl.BlockSpec((1, 1, H), lambda i, g, a: (g[i], 0, 0)),
        ],
        out_specs=pl.BlockSpec((TILE, H), lambda i, g, a: (i, 0)),
    )
    return pl.pallas_call(
        _gmm_body,
        grid_spec=grid_spec,
        out_shape=jax.ShapeDtypeStruct((P, H), jnp.float32),
    )(gid, actv, xg, wg, gub3, wdp, db3)


def _combine_body(y_hbm, pos_hbm, w0_hbm, w1_hbm, out_hbm,
                  buf0, buf1, idx0_v, idx1_v, w0_v, w1_v, sem0, sem1):
    wid = lax.axis_index("s") * 2 + lax.axis_index("c")

    @pl.loop(0, TOK_PER_SUB // CCHUNK)
    def _(ci):
        tok = wid * TOK_PER_SUB + ci * CCHUNK
        pltpu.sync_copy(pos_hbm.at[0, pl.ds(tok, CCHUNK)], idx0_v)
        pltpu.sync_copy(pos_hbm.at[1, pl.ds(tok, CCHUNK)], idx1_v)
        cp0 = pltpu.async_copy(y_hbm.at[idx0_v], buf0, sem0)
        cp1 = pltpu.async_copy(y_hbm.at[idx1_v], buf1, sem1)
        pltpu.sync_copy(w0_hbm.at[pl.ds(tok, CCHUNK)], w0_v)
        pltpu.sync_copy(w1_hbm.at[pl.ds(tok, CCHUNK)], w1_v)
        cp0.wait()
        cp1.wait()

        @pl.loop(0, CCHUNK)
        def _(t):
            w0 = w0_v[t, :]
            w1 = w1_v[t, :]

            @pl.loop(0, H, step=16)
            def _(h):
                buf0[t, pl.ds(h, 16)] = (
                    w0 * buf0[t, pl.ds(h, 16)] + w1 * buf1[t, pl.ds(h, 16)])

        pltpu.sync_copy(buf0, out_hbm.at[pl.ds(tok, CCHUNK)])


def _combine_call(y, pos2, w0w, w1w):
    mesh = plsc.VectorSubcoreMesh(core_axis_name="c", subcore_axis_name="s")
    kern = pl.kernel(
        _combine_body,
        out_type=jax.ShapeDtypeStruct((S, H), jnp.float32),
        mesh=mesh,
        scratch_types=[
            pltpu.VMEM((CCHUNK, H), jnp.float32),
            pltpu.VMEM((CCHUNK, H), jnp.float32),
            pltpu.VMEM((CCHUNK,), jnp.int32),
            pltpu.VMEM((CCHUNK,), jnp.int32),
            pltpu.VMEM((CCHUNK, 16), jnp.float32),
            pltpu.VMEM((CCHUNK, 16), jnp.float32),
            pltpu.SemaphoreType.DMA,
            pltpu.SemaphoreType.DMA,
        ],
    )
    return kern(y, pos2, w0w, w1w)


@jax.jit
def _kernel_impl(hidden_states, router_weight, router_bias, gate_up_proj,
                 gate_up_bias, down_proj, down_bias):
    batch, seq, hid = hidden_states.shape
    flat = hidden_states.reshape(batch * seq, hid)
    wrt = router_weight.T
    bias2d = router_bias.reshape(1, E)

    pos, w0w, w1w, gid, actv = _meta_call(flat, wrt, bias2d)
    pos3 = pos.reshape(K, NSUB, TOK_PER_SUB)
    gid1 = gid.reshape(NTPAD)
    actv1 = actv.reshape(NTPAD)

    xg = _dispatch_call(flat, pos3)

    gub3 = gate_up_bias.reshape(E, 1, 2 * I)
    db3 = down_bias.reshape(E, 1, H)
    y = _gmm_call(xg, gate_up_proj, gub3, down_proj, db3, gid1, actv1)

    pos2 = pos.reshape(K, S)
    out = _combine_call(y, pos2, w0w, w1w)
    return out.reshape(batch, seq, hid)


def kernel(hidden_states, router_weight, router_bias, gate_up_proj,
           gate_up_bias, down_proj, down_bias):
    return _kernel_impl(hidden_states, router_weight, router_bias,
                        gate_up_proj, gate_up_bias, down_proj, down_bias)
